# baseline (speedup 1.0000x reference)
import functools
import math

import jax
import jax.numpy as jnp
from jax import lax
from jax.experimental import pallas as pl
from jax.experimental.pallas import tpu as pltpu
from jax.experimental.pallas import tpu_sc as plsc

B, S, H, G, K = 16, 1, 16, 1, 2048
DIM, TAIL = 512, 64
D = DIM + TAIL
SKV = 8192
SM_SCALE = 1.0 / math.sqrt(D)

NC, NS = 2, 16
LANES = 16


def _sc_histogram(idx2d):
    nb = idx2d.shape[0]
    mesh = plsc.VectorSubcoreMesh(core_axis_name="c", subcore_axis_name="s")

    @functools.partial(
        pl.kernel,
        out_type=jax.ShapeDtypeStruct((nb, 1, SKV), jnp.float32),
        mesh=mesh,
        scratch_types=[
            pltpu.VMEM((K,), jnp.int32),
            pltpu.VMEM((SKV,), jnp.float32),
        ],
        compiler_params=pltpu.CompilerParams(
            use_tc_tiling_on_sc=False, needs_layout_passes=False),
    )
    def hist_kernel(idx_hbm, out_hbm, idx_v, cnt_v):
        wid = lax.axis_index("s") * NC + lax.axis_index("c")

        @pl.when(wid < nb)
        def _():
            pltpu.sync_copy(idx_hbm.at[wid], idx_v)
            zeros = jnp.zeros((LANES,), jnp.float32)
            for j in range(SKV // LANES):
                cnt_v[pl.ds(j * LANES, LANES)] = zeros
            ones = jnp.ones((LANES,), jnp.float32)
            for j in range(K // LANES):
                ids = idx_v[pl.ds(j * LANES, LANES)]
                plsc.addupdate_scatter(cnt_v, [ids], ones)
            pltpu.sync_copy(cnt_v, out_hbm.at[wid, 0])

    return hist_kernel(idx2d)


def _attn_body(q_ref, kvt_ref, cnt_ref, o_ref):
    qb = q_ref[0]
    kt = kvt_ref[0]
    cnt = cnt_ref[0]
    s = lax.dot_general(qb, kt, (((1,), (0,)), ((), ())),
                        preferred_element_type=jnp.float32) * SM_SCALE
    m = jnp.max(s, axis=1, keepdims=True)
    e = jnp.exp(s - m) * cnt
    den = jnp.sum(e, axis=1, keepdims=True)
    o = lax.dot_general(e, kt[:DIM, :], (((1,), (1,)), ((), ())),
                        preferred_element_type=jnp.float32)
    o_ref[0] = o / den


def _tc_attention(q3, kvt3, counts):
    nb = q3.shape[0]
    return pl.pallas_call(
        _attn_body,
        grid=(nb,),
        in_specs=[
            pl.BlockSpec((1, H, D), lambda b: (b, 0, 0)),
            pl.BlockSpec((1, D, SKV), lambda b: (b, 0, 0)),
            pl.BlockSpec((1, 1, SKV), lambda b: (b, 0, 0)),
        ],
        out_specs=pl.BlockSpec((1, H, DIM), lambda b: (b, 0, 0)),
        out_shape=jax.ShapeDtypeStruct((nb, H, DIM), jnp.float32),
    )(q3, kvt3, counts)


def kernel(q, kv, indices):
    idx2 = indices.reshape(B, K)
    kvt3 = jnp.transpose(kv, (0, 2, 3, 1)).reshape(B, D, SKV)
    q3 = q.reshape(B, H, D)
    hb = B // 2
    counts_a = _sc_histogram(idx2[:hb])
    counts_b = _sc_histogram(idx2[hb:])
    out_a = _tc_attention(q3[:hb], kvt3[:hb], counts_a)
    out_b = _tc_attention(q3[hb:], kvt3[hb:], counts_b)
    out = jnp.concatenate([out_a, out_b], axis=0)
    return out.reshape(B, S, H, DIM)


# speedup vs baseline: 2.6469x; 2.6469x over previous
import functools
import math

import jax
import jax.numpy as jnp
from jax import lax
from jax.experimental import pallas as pl
from jax.experimental.pallas import tpu as pltpu
from jax.experimental.pallas import tpu_sc as plsc

B, S, H, G, K = 16, 1, 16, 1, 2048
DIM, TAIL = 512, 64
D = DIM + TAIL
SKV = 8192
SM_SCALE = 1.0 / math.sqrt(D)

NC, NS = 2, 16
LANES = 16


def _sc_histogram(idx2d):
    mesh = plsc.VectorSubcoreMesh(core_axis_name="c", subcore_axis_name="s")

    @functools.partial(
        pl.kernel,
        out_type=jax.ShapeDtypeStruct((B, 1, SKV), jnp.float32),
        mesh=mesh,
        scratch_types=[
            pltpu.VMEM((K,), jnp.int32),
            pltpu.VMEM((SKV,), jnp.float32),
        ],
        compiler_params=pltpu.CompilerParams(
            use_tc_tiling_on_sc=False, needs_layout_passes=False),
    )
    def hist_kernel(idx_hbm, out_hbm, idx_v, cnt_v):
        wid = lax.axis_index("s") * NC + lax.axis_index("c")

        @pl.when(wid < B)
        def _():
            pltpu.sync_copy(idx_hbm.at[wid], idx_v)
            zeros = jnp.zeros((LANES,), jnp.float32)
            for j in range(SKV // LANES):
                cnt_v[pl.ds(j * LANES, LANES)] = zeros
            ones = jnp.ones((LANES,), jnp.float32)
            for j in range(K // LANES):
                ids = idx_v[pl.ds(j * LANES, LANES)]
                plsc.addupdate_scatter(cnt_v, [ids], ones)
            pltpu.sync_copy(cnt_v, out_hbm.at[wid, 0])

    return hist_kernel(idx2d)


def _attn_body(q_ref, kvt_ref, cnt_ref, o_ref):
    qb = q_ref[0]
    kt = kvt_ref[0]
    cnt = cnt_ref[0]
    s = lax.dot_general(qb, kt, (((1,), (0,)), ((), ())),
                        preferred_element_type=jnp.float32) * SM_SCALE
    m = jnp.max(s, axis=1, keepdims=True)
    e = jnp.exp(s - m) * cnt
    den = jnp.sum(e, axis=1, keepdims=True)
    o = lax.dot_general(e, kt[:DIM, :], (((1,), (1,)), ((), ())),
                        preferred_element_type=jnp.float32)
    o_ref[0] = o / den


def _tc_attention(q3, kvt3, counts):
    return pl.pallas_call(
        _attn_body,
        grid=(B,),
        in_specs=[
            pl.BlockSpec((1, H, D), lambda b: (b, 0, 0)),
            pl.BlockSpec((1, D, SKV), lambda b: (b, 0, 0)),
            pl.BlockSpec((1, 1, SKV), lambda b: (b, 0, 0)),
        ],
        out_specs=pl.BlockSpec((1, H, DIM), lambda b: (b, 0, 0)),
        out_shape=jax.ShapeDtypeStruct((B, H, DIM), jnp.float32),
    )(q3, kvt3, counts)


def kernel(q, kv, indices):
    counts = _sc_histogram(indices.reshape(B, K))
    kvt3 = jnp.transpose(kv, (0, 2, 3, 1)).reshape(B, D, SKV)
    out = _tc_attention(q.reshape(B, H, D), kvt3, counts)
    return out.reshape(B, S, H, DIM)
